# Initial kernel scaffold; baseline (speedup 1.0000x reference)
import jax
import jax.numpy as jnp
from jax.experimental import pallas as pl


def kernel(x, s0_w0, s0_b0, s0_w1, s0_b1, s0_gamma, s0_beta, s0_w2, s0_b2, s1_w0, s1_b0, s1_w1, s1_b1, s1_gamma, s1_beta, s1_w2, s1_b2, s2_w0, s2_b0, s2_w1, s2_b1, s2_gamma, s2_beta, s2_w2, s2_b2):
    raise NotImplementedError("write your pallas kernel here")


# speedup vs baseline: 1.0291x; 1.0291x over previous
import functools

import jax
import jax.numpy as jnp
from jax.experimental import pallas as pl
from jax.experimental.pallas import tpu as pltpu

_SLOPE = 0.2
_EPS = 1e-5


def _lrelu(v):
    return jnp.where(v >= 0, v, _SLOPE * v)


def _up8(v):
    return (v + 7) // 8 * 8


def _mm(col, w):
    H, Wp, K = col.shape
    out = jnp.dot(col.reshape(H * Wp, K), w,
                  preferred_element_type=jnp.float32)
    return out.reshape(H, Wp, w.shape[1])


def _front_one(x_ref, w0_ref, b0_ref, w1_ref, b1_ref,
               d0_ref, c1_ref, ssum_ref, ssq_ref,
               colb, pad0, cola, dims):
    H0, W0, H1, W1, CinE, C0 = dims
    for b in range(2):
        colb[:, pl.ds(0, W0), b * CinE:(b + 1) * CinE] = (
            x_ref[0, :, pl.ds(b, W0), :])
    d0 = _mm(colb[pl.ds(0, H0)], w0_ref[0])
    d0 = d0 + _mm(colb[pl.ds(1, H0)], w0_ref[1])
    d0 = _lrelu(d0 + b0_ref[...])[:, :W0, :]
    d0_ref[0] = d0

    pad0[...] = jnp.zeros(pad0.shape, pad0.dtype)
    pad0[2:2 + H0, 2:2 + W0, :] = d0
    for kw in range(4):
        cola[:, pl.ds(0, W1), kw * C0:(kw + 1) * C0] = pad0[:, pl.ds(kw, W1), :]
    c1 = _mm(cola[pl.ds(0, H1)], w1_ref[0])
    for kh in range(1, 4):
        c1 = c1 + _mm(cola[pl.ds(kh, H1)], w1_ref[kh])
    c1 = c1[:, :W1, :] + b1_ref[...]
    c1_ref[0] = c1.astype(jnp.bfloat16)
    ssum_ref[...] = jnp.sum(c1, axis=(0, 1), keepdims=True)
    ssq_ref[...] = jnp.sum(c1 * c1, axis=(0, 1), keepdims=True)


def _front_all(*refs, dims3):
    for i in range(3):
        _front_one(*refs[5 * i:5 * i + 5],
                   *refs[15 + 4 * i:15 + 4 * i + 4],
                   *refs[27 + 3 * i:27 + 3 * i + 3], dims3[i])


def _back_one(c1_ref, ssum_ref, ssq_ref, g_ref, bt_ref, w2_ref, b2_ref,
              d1_ref, d2_ref, pad1, colc, dims):
    H1, W1, H2, W2, C1, count = dims
    mean = jnp.sum(ssum_ref[...], axis=0) * (1.0 / count)
    var = jnp.sum(ssq_ref[...], axis=0) * (1.0 / count) - mean * mean
    scl = g_ref[...] * jax.lax.rsqrt(var + _EPS)
    shf = bt_ref[...] - mean * scl

    y = _lrelu(c1_ref[0].astype(jnp.float32) * scl + shf)
    d1_ref[0] = y

    pad1[...] = jnp.zeros(pad1.shape, pad1.dtype)
    pad1[2:2 + H1, 2:2 + W1, :] = y
    for kw in range(4):
        colc[:, pl.ds(0, W2), kw * C1:(kw + 1) * C1] = pad1[:, pl.ds(kw, W2), :]
    acc = _mm(colc[pl.ds(0, H2)], w2_ref[0])
    for kh in range(1, 4):
        acc = acc + _mm(colc[pl.ds(kh, H2)], w2_ref[kh])
    d2_ref[0] = acc[:, :W2, 0] + b2_ref[...]


def _back_all(*refs, dims3):
    for i in range(3):
        _back_one(*refs[7 * i:7 * i + 7],
                  *refs[21 + 2 * i:21 + 2 * i + 2],
                  *refs[27 + 2 * i:27 + 2 * i + 2], dims3[i])


def _pool_body(p00_ref, p01_ref, p10_ref, p11_ref, inv_ref, o_ref, *, Ho, Wo):
    ph = (p00_ref, p01_ref, p10_ref, p11_ref)
    acc = jnp.zeros(o_ref.shape[1:], jnp.float32)
    for kh in range(3):
        a, r = kh // 2, kh % 2
        for kw in range(3):
            b, s = kw // 2, kw % 2
            acc = acc + ph[2 * r + s][0, pl.ds(a, Ho), pl.ds(b, Wo), :]
    o_ref[0] = acc * inv_ref[...]


def _avgpool(x):
    N, H, W, C = x.shape
    Ho, Wo = (H - 1) // 2 + 1, (W - 1) // 2 + 1
    xp = jnp.pad(x, ((0, 0), (1, 2), (1, 2), (0, 0)))
    phases = [xp[:, r:r + 2 * Ho + 1:2, s:s + 2 * Wo + 1:2, :]
              for r in (0, 1) for s in (0, 1)]

    ri = 2 * jnp.arange(Ho)[:, None] - 1 + jnp.arange(3)[None, :]
    ci = 2 * jnp.arange(Wo)[:, None] - 1 + jnp.arange(3)[None, :]
    rcnt = jnp.sum((ri >= 0) & (ri < H), axis=1).astype(jnp.float32)
    ccnt = jnp.sum((ci >= 0) & (ci < W), axis=1).astype(jnp.float32)
    inv = (1.0 / (rcnt[:, None] * ccnt[None, :]))[:, :, None]

    Hh, Wh = Ho + 1, Wo + 1
    body = functools.partial(_pool_body, Ho=Ho, Wo=Wo)
    return pl.pallas_call(
        body,
        out_shape=jax.ShapeDtypeStruct((N, Ho, Wo, C), jnp.float32),
        grid=(N,),
        in_specs=[pl.BlockSpec((1, Hh, Wh, C), lambda n: (n, 0, 0, 0))] * 4
                 + [pl.BlockSpec((Ho, Wo, 1), lambda n: (0, 0, 0))],
        out_specs=pl.BlockSpec((1, Ho, Wo, C), lambda n: (n, 0, 0, 0)),
        compiler_params=pltpu.CompilerParams(dimension_semantics=("parallel",)),
    )(*phases, inv)


def _prep_scale(x, w0, w1):
    N, H, W, Cin = x.shape
    C0, C1 = w0.shape[-1], w1.shape[-1]
    H0, W0 = H // 2 + 1, W // 2 + 1
    CinE = 4 * Cin
    xp = jnp.pad(x, ((0, 0), (2, 2), (2, 2), (0, 0)))
    xs2d = jnp.concatenate(
        [xp[:, r:r + 2 * H0 + 1:2, s:s + 2 * W0 + 1:2, :]
         for r in (0, 1) for s in (0, 1)], axis=-1)
    w0g = jnp.transpose(w0.reshape(2, 2, 2, 2, Cin, C0),
                        (0, 2, 1, 3, 4, 5)).reshape(2, 2 * CinE, C0)
    w1g = w1.reshape(4, 4 * C0, C1)
    return xs2d, w0g, w1g


def kernel(x, s0_w0, s0_b0, s0_w1, s0_b1, s0_gamma, s0_beta, s0_w2, s0_b2,
           s1_w0, s1_b0, s1_w1, s1_b1, s1_gamma, s1_beta, s1_w2, s1_b2,
           s2_w0, s2_b0, s2_w1, s2_b1, s2_gamma, s2_beta, s2_w2, s2_b2):
    xh = jnp.transpose(x, (0, 2, 3, 1)).astype(jnp.float32)
    N = x.shape[0]
    params = [
        dict(w0=s2_w0, b0=s2_b0, w1=s2_w1, b1=s2_b1, gamma=s2_gamma,
             beta=s2_beta, w2=s2_w2, b2=s2_b2),
        dict(w0=s1_w0, b0=s1_b0, w1=s1_w1, b1=s1_b1, gamma=s1_gamma,
             beta=s1_beta, w2=s1_w2, b2=s1_b2),
        dict(w0=s0_w0, b0=s0_b0, w1=s0_w1, b1=s0_b1, gamma=s0_gamma,
             beta=s0_beta, w2=s0_w2, b2=s0_b2),
    ]
    xs = [xh, _avgpool(xh)]
    xs.append(_avgpool(xs[1]))

    geo = []
    front_in, front_in_specs = [], []
    front_out_shape, front_out_specs, front_scratch = [], [], []
    fdims = []
    for xi, p in zip(xs, params):
        _, H, W, Cin = xi.shape
        C0, C1 = p["w0"].shape[-1], p["w1"].shape[-1]
        H0, W0 = H // 2 + 1, W // 2 + 1
        H1, W1 = H0 + 1, W0 + 1
        CinE = 4 * Cin
        geo.append((H0, W0, H1, W1, CinE, C0, C1))
        fdims.append((H0, W0, H1, W1, CinE, C0))
        xs2d, w0g, w1g = _prep_scale(xi, p["w0"], p["w1"])
        front_in += [xs2d, w0g, p["b0"].reshape(1, C0), w1g,
                     p["b1"].reshape(1, C1)]
        front_in_specs += [
            pl.BlockSpec((1, H0 + 1, W0 + 1, CinE), lambda n: (n, 0, 0, 0)),
            pl.BlockSpec((2, 2 * CinE, C0), lambda n: (0, 0, 0)),
            pl.BlockSpec((1, C0), lambda n: (0, 0)),
            pl.BlockSpec((4, 4 * C0, C1), lambda n: (0, 0, 0)),
            pl.BlockSpec((1, C1), lambda n: (0, 0))]
        front_out_shape += [
            jax.ShapeDtypeStruct((N, H0, W0, C0), jnp.float32),
            jax.ShapeDtypeStruct((N, H1, W1, C1), jnp.bfloat16),
            jax.ShapeDtypeStruct((N, 1, C1), jnp.float32),
            jax.ShapeDtypeStruct((N, 1, C1), jnp.float32)]
        front_out_specs += [
            pl.BlockSpec((1, H0, W0, C0), lambda n: (n, 0, 0, 0)),
            pl.BlockSpec((1, H1, W1, C1), lambda n: (n, 0, 0, 0)),
            pl.BlockSpec((1, 1, C1), lambda n: (n, 0, 0)),
            pl.BlockSpec((1, 1, C1), lambda n: (n, 0, 0))]
        front_scratch += [
            pltpu.VMEM((H0 + 1, _up8(W0), 2 * CinE), jnp.float32),
            pltpu.VMEM((H0 + 4, _up8(W0 + 4), C0), jnp.float32),
            pltpu.VMEM((H1 + 3, _up8(W1), 4 * C0), jnp.float32)]

    front = functools.partial(_front_all, dims3=fdims)
    fouts = pl.pallas_call(
        front,
        out_shape=tuple(front_out_shape),
        grid=(N,),
        in_specs=front_in_specs,
        out_specs=tuple(front_out_specs),
        scratch_shapes=front_scratch,
        compiler_params=pltpu.CompilerParams(dimension_semantics=("parallel",)),
    )(*front_in)

    back_in, back_in_specs = [], []
    back_out_shape, back_out_specs, back_scratch = [], [], []
    bdims = []
    for i, p in enumerate(params):
        H0, W0, H1, W1, CinE, C0, C1 = geo[i]
        H2, W2 = H1 + 1, W1 + 1
        d0, c1, ssum, ssq = fouts[4 * i:4 * i + 4]
        bdims.append((H1, W1, H2, W2, C1, float(N * H1 * W1)))
        back_in += [c1, ssum, ssq, p["gamma"].reshape(1, C1),
                    p["beta"].reshape(1, C1), p["w2"].reshape(4, 4 * C1, 1),
                    p["b2"].reshape(1, 1)]
        back_in_specs += [
            pl.BlockSpec((1, H1, W1, C1), lambda n: (n, 0, 0, 0)),
            pl.BlockSpec((N, 1, C1), lambda n: (0, 0, 0)),
            pl.BlockSpec((N, 1, C1), lambda n: (0, 0, 0)),
            pl.BlockSpec((1, C1), lambda n: (0, 0)),
            pl.BlockSpec((1, C1), lambda n: (0, 0)),
            pl.BlockSpec((4, 4 * C1, 1), lambda n: (0, 0, 0)),
            pl.BlockSpec((1, 1), lambda n: (0, 0))]
        back_out_shape += [
            jax.ShapeDtypeStruct((N, H1, W1, C1), jnp.float32),
            jax.ShapeDtypeStruct((N, H2, W2), jnp.float32)]
        back_out_specs += [
            pl.BlockSpec((1, H1, W1, C1), lambda n: (n, 0, 0, 0)),
            pl.BlockSpec((1, H2, W2), lambda n: (n, 0, 0))]
        back_scratch += [
            pltpu.VMEM((H1 + 4, _up8(W1 + 4), C1), jnp.float32),
            pltpu.VMEM((H2 + 3, _up8(W2), 4 * C1), jnp.float32)]

    back = functools.partial(_back_all, dims3=bdims)
    bouts = pl.pallas_call(
        back,
        out_shape=tuple(back_out_shape),
        grid=(N,),
        in_specs=back_in_specs,
        out_specs=tuple(back_out_specs),
        scratch_shapes=back_scratch,
        compiler_params=pltpu.CompilerParams(dimension_semantics=("parallel",)),
    )(*back_in)

    nchw = lambda t: jnp.transpose(t, (0, 3, 1, 2))
    results = []
    for i in range(3):
        d0 = fouts[4 * i]
        d1, d2 = bouts[2 * i:2 * i + 2]
        results.append([nchw(d0), nchw(d1), d2[:, None, :, :]])
    return results
